# Initial kernel scaffold; baseline (speedup 1.0000x reference)
import jax
import jax.numpy as jnp
from jax.experimental import pallas as pl


def kernel(x, w1, b1, w2, b2, gamma, beta):
    raise NotImplementedError("write your pallas kernel here")


# speedup vs baseline: 1.2899x; 1.2899x over previous
import functools

import jax
import jax.numpy as jnp
from jax.experimental import pallas as pl
from jax.experimental.pallas import tpu as pltpu


def _round_up(n, m):
    return ((n + m - 1) // m) * m


def _fused_ffn_ln(x_ref, w1_ref, b1_ref, w2_ref, b2_ref, g_ref, bt_ref,
                  o_ref, *, eps, inv_d):
    x32 = x_ref[...]
    h = jnp.dot(x32.astype(jnp.bfloat16), w1_ref[...],
                preferred_element_type=jnp.float32)
    h = jnp.maximum(h + b1_ref[...], 0.0)
    y = jnp.dot(h.astype(jnp.bfloat16), w2_ref[...],
                preferred_element_type=jnp.float32)
    y = y + b2_ref[...] + x32
    mean = jnp.sum(y, axis=-1, keepdims=True) * inv_d
    var = jnp.sum(y * y, axis=-1, keepdims=True) * inv_d - mean * mean
    var = jnp.maximum(var, 0.0)
    o_ref[...] = (y - mean) * jax.lax.rsqrt(var + eps) * g_ref[...] + bt_ref[...]


def kernel(x, w1, b1, w2, b2, gamma, beta, *, eps=1e-6, tile_rows=256):
    B, S, d_in = x.shape
    d_hid = w1.shape[1]
    N = B * S

    d_in_p = _round_up(d_in, 128)
    d_hid_p = _round_up(d_hid, 128)
    N_p = _round_up(N, tile_rows)

    x2 = x.reshape(N, d_in)
    if N_p != N or d_in_p != d_in:
        x2 = jnp.pad(x2, ((0, N_p - N), (0, d_in_p - d_in)))
    if d_in_p != d_in or d_hid_p != d_hid:
        w1 = jnp.pad(w1, ((0, d_in_p - d_in), (0, d_hid_p - d_hid)))
        w2 = jnp.pad(w2, ((0, d_hid_p - d_hid), (0, d_in_p - d_in)))
        b1 = jnp.pad(b1, (0, d_hid_p - d_hid))
        b2 = jnp.pad(b2, (0, d_in_p - d_in))
        gamma = jnp.pad(gamma, (0, d_in_p - d_in))
        beta = jnp.pad(beta, (0, d_in_p - d_in))

    w1b = w1.astype(jnp.bfloat16)
    w2b = w2.astype(jnp.bfloat16)
    b1r = b1.reshape(1, d_hid_p)
    b2r = b2.reshape(1, d_in_p)
    gr = gamma.reshape(1, d_in_p)
    br = beta.reshape(1, d_in_p)

    n_row_tiles = N_p // tile_rows

    weight_bytes = (w1b.size + w2b.size) * 2 + (b1r.size + 3 * b2r.size) * 4
    cost = pl.CostEstimate(
        flops=4 * N_p * d_in_p * d_hid_p,
        transcendentals=N_p,
        bytes_accessed=2 * N_p * d_in_p * 4 + weight_bytes,
    )

    out = pl.pallas_call(
        functools.partial(_fused_ffn_ln, eps=eps, inv_d=1.0 / d_in),
        out_shape=jax.ShapeDtypeStruct((N_p, d_in_p), x.dtype),
        grid=(n_row_tiles,),
        in_specs=[
            pl.BlockSpec((tile_rows, d_in_p), lambda i: (i, 0)),
            pl.BlockSpec((d_in_p, d_hid_p), lambda i: (0, 0)),
            pl.BlockSpec((1, d_hid_p), lambda i: (0, 0)),
            pl.BlockSpec((d_hid_p, d_in_p), lambda i: (0, 0)),
            pl.BlockSpec((1, d_in_p), lambda i: (0, 0)),
            pl.BlockSpec((1, d_in_p), lambda i: (0, 0)),
            pl.BlockSpec((1, d_in_p), lambda i: (0, 0)),
        ],
        out_specs=pl.BlockSpec((tile_rows, d_in_p), lambda i: (i, 0)),
        compiler_params=pltpu.CompilerParams(
            dimension_semantics=("parallel",),
            vmem_limit_bytes=int((64 << 20) * 0.98),
        ),
        cost_estimate=cost,
    )(x2, w1b, b1r, w2b, b2r, gr, br)

    if N_p != N or d_in_p != d_in:
        out = out[:N, :d_in]
    return out.reshape(B, S, d_in)
